# Initial kernel scaffold; baseline (speedup 1.0000x reference)
import jax
import jax.numpy as jnp
from jax.experimental import pallas as pl


def kernel(x, edge_index, rel_repr, edge_type, edge_norm, in_w, out_w, loop_w, w_rel, loop_rel, bias, bn_gamma, bn_beta):
    raise NotImplementedError("write your pallas kernel here")


# speedup vs baseline: 2.9596x; 2.9596x over previous
import functools

import jax
import jax.numpy as jnp
from jax import lax
from jax.experimental import pallas as pl
from jax.experimental.pallas import tpu as pltpu
from jax.experimental.pallas import tpu_sc as plsc

_NC = 2
_NS = 16
_CHUNK = 80


def _sc_aggregate(x, src, dst, etype, norm, rel, zeros):
    n, d = x.shape
    e = src.shape[0]
    half = e // 2
    per_w = half // _NS
    n_chunks = per_w // _CHUNK
    rows_w = -(-n // (_NS * 8)) * 8
    n_pad = rows_w * _NS

    mesh = plsc.VectorSubcoreMesh(core_axis_name="c", subcore_axis_name="s",
                                  num_cores=_NC, num_subcores=_NS)

    @functools.partial(
        pl.kernel,
        out_type=jax.ShapeDtypeStruct((_NC, n_pad, d), jnp.float32),
        mesh=mesh,
        compiler_params=pltpu.CompilerParams(needs_layout_passes=False),
        scratch_types=[
            pltpu.VMEM((_CHUNK,), jnp.int32),
            pltpu.VMEM((_CHUNK,), jnp.int32),
            pltpu.VMEM((_CHUNK,), jnp.int32),
            pltpu.VMEM((_CHUNK,), jnp.float32),
            pltpu.VMEM((_CHUNK, d), jnp.float32),
            pltpu.VMEM((_CHUNK, d), jnp.float32),
            pltpu.VMEM_SHARED((n_pad, d), jnp.float32),
            pltpu.SemaphoreType.DMA,
            pltpu.SemaphoreType.DMA,
        ],
    )
    def agg(x_hbm, src_hbm, dst_hbm, et_hbm, norm_hbm, rel_hbm, zeros_hbm,
            out_hbm, src_v, et_v, dst_v, norm_v, xrows_v, relrows_v,
            acc_sh, sem1, sem2):
        c = lax.axis_index("c")
        s = lax.axis_index("s")
        row0 = s * rows_w

        pltpu.sync_copy(zeros_hbm, acc_sh.at[pl.ds(row0, rows_w)])
        plsc.subcore_barrier()

        base = c * half + s * per_w

        def chunk_body(j, carry):
            off = base + j * _CHUNK
            pltpu.sync_copy(src_hbm.at[pl.ds(off, _CHUNK)], src_v)
            pltpu.sync_copy(et_hbm.at[pl.ds(off, _CHUNK)], et_v)
            pltpu.sync_copy(dst_hbm.at[pl.ds(off, _CHUNK)], dst_v)
            pltpu.sync_copy(norm_hbm.at[pl.ds(off, _CHUNK)], norm_v)
            cp1 = pltpu.async_copy(x_hbm.at[src_v], xrows_v, sem1)
            cp2 = pltpu.async_copy(rel_hbm.at[et_v], relrows_v, sem2)
            cp1.wait()
            cp2.wait()

            def edge_body(i, carry2):
                nb = plsc.load_gather(norm_v, [jnp.full((16,), i, jnp.int32)])
                for q in range(d // 16):
                    sl = pl.ds(q * 16, 16)
                    xrows_v[i, sl] = xrows_v[i, sl] * relrows_v[i, sl] * nb
                return carry2

            lax.fori_loop(0, _CHUNK, edge_body, 0)
            pltpu.sync_copy(xrows_v, acc_sh.at[dst_v], add=True)
            return carry

        lax.fori_loop(0, n_chunks, chunk_body, 0)

        plsc.subcore_barrier()
        pltpu.sync_copy(acc_sh.at[pl.ds(row0, rows_w)],
                        out_hbm.at[c, pl.ds(row0, rows_w)])

    return agg(x, src, dst, etype, norm, rel, zeros)


_BLK = 2000


def _tc_stage1(acc_ref, x_ref, inw_ref, outw_ref, loopw_ref, wrel_ref,
               looprel_ref, bias_ref, rel_ref, h_ref, stats_ref, relout_ref):
    i = pl.program_id(0)
    hp = lax.Precision.HIGHEST
    f32 = jnp.float32
    h = jnp.dot(acc_ref[0], inw_ref[:], precision=hp,
                preferred_element_type=f32)
    h = h + jnp.dot(acc_ref[1], outw_ref[:], precision=hp,
                    preferred_element_type=f32)
    h = h + jnp.dot(x_ref[:] * looprel_ref[:], loopw_ref[:], precision=hp,
                    preferred_element_type=f32)
    h = h * (1.0 / 3.0) + bias_ref[:]
    h_ref[:] = h

    @pl.when(i == 0)
    def _():
        stats_ref[:] = jnp.zeros_like(stats_ref)
        relout_ref[:] = jnp.dot(rel_ref[:], wrel_ref[:], precision=hp,
                                preferred_element_type=f32)

    stats_ref[0:1, :] += jnp.sum(h, axis=0, keepdims=True)
    stats_ref[1:2, :] += jnp.sum(h * h, axis=0, keepdims=True)


def _tc_stage2(h_ref, stats_ref, gamma_ref, beta_ref, out_ref, *, n):
    mean = stats_ref[0:1, :] * (1.0 / n)
    ex2 = stats_ref[1:2, :] * (1.0 / n)
    var = ex2 - mean * mean
    out_ref[:] = ((h_ref[:] - mean)
                  * (gamma_ref[:] * lax.rsqrt(var + 1e-5)) + beta_ref[:])


def kernel(x, edge_index, rel_repr, edge_type, edge_norm, in_w, out_w, loop_w,
           w_rel, loop_rel, bias, bn_gamma, bn_beta):
    n, d = x.shape
    nr = rel_repr.shape[0]
    src = edge_index[0]
    dst = edge_index[1]
    zeros = jnp.zeros((-(-n // (_NS * 8)) * 8, d), jnp.float32)
    acc = _sc_aggregate(x, src, dst, edge_type, edge_norm, rel_repr, zeros)

    grid = n // _BLK
    const = lambda *_: tuple(0 for _ in range(2))
    h, stats, relout = pl.pallas_call(
        _tc_stage1,
        grid=(grid,),
        in_specs=[
            pl.BlockSpec((2, _BLK, d), lambda i: (0, i, 0)),
            pl.BlockSpec((_BLK, d), lambda i: (i, 0)),
            pl.BlockSpec((d, d), const),
            pl.BlockSpec((d, d), const),
            pl.BlockSpec((d, d), const),
            pl.BlockSpec((d, d), const),
            pl.BlockSpec((1, d), const),
            pl.BlockSpec((1, d), const),
            pl.BlockSpec((nr, d), const),
        ],
        out_specs=[
            pl.BlockSpec((_BLK, d), lambda i: (i, 0)),
            pl.BlockSpec((8, d), const),
            pl.BlockSpec((nr, d), const),
        ],
        out_shape=(jax.ShapeDtypeStruct((n, d), jnp.float32),
                   jax.ShapeDtypeStruct((8, d), jnp.float32),
                   jax.ShapeDtypeStruct((nr, d), jnp.float32)),
    )(acc, x, in_w, out_w, loop_w, w_rel, loop_rel,
      bias.reshape(1, d), rel_repr)

    out = pl.pallas_call(
        functools.partial(_tc_stage2, n=n),
        grid=(grid,),
        in_specs=[
            pl.BlockSpec((_BLK, d), lambda i: (i, 0)),
            pl.BlockSpec((8, d), const),
            pl.BlockSpec((1, d), const),
            pl.BlockSpec((1, d), const),
        ],
        out_specs=pl.BlockSpec((_BLK, d), lambda i: (i, 0)),
        out_shape=jax.ShapeDtypeStruct((n, d), jnp.float32),
    )(h, stats, bn_gamma.reshape(1, d), bn_beta.reshape(1, d))
    return out, relout


# speedup vs baseline: 4.4088x; 1.4897x over previous
import functools

import jax
import jax.numpy as jnp
from jax import lax
from jax.experimental import pallas as pl
from jax.experimental.pallas import tpu as pltpu
from jax.experimental.pallas import tpu_sc as plsc

_NC = 2
_NS = 16
_CHUNK = 80


def _sc_aggregate(x, src, dst, etype, norm, rel, zeros):
    n, d = x.shape
    e = src.shape[0]
    half = e // 2
    per_w = half // _NS
    n_chunks = per_w // _CHUNK
    rows_w = -(-n // (_NS * 8)) * 8
    n_pad = rows_w * _NS

    mesh = plsc.VectorSubcoreMesh(core_axis_name="c", subcore_axis_name="s",
                                  num_cores=_NC, num_subcores=_NS)

    @functools.partial(
        pl.kernel,
        out_type=jax.ShapeDtypeStruct((_NC, n_pad, d), jnp.float32),
        mesh=mesh,
        compiler_params=pltpu.CompilerParams(needs_layout_passes=False),
        scratch_types=[
            [pltpu.VMEM((_CHUNK,), jnp.int32)] * 2,
            [pltpu.VMEM((_CHUNK,), jnp.int32)] * 2,
            [pltpu.VMEM((_CHUNK,), jnp.int32)] * 2,
            [pltpu.VMEM((_CHUNK,), jnp.float32)] * 2,
            [pltpu.VMEM((_CHUNK, d), jnp.float32)] * 2,
            [pltpu.VMEM((_CHUNK, d), jnp.float32)] * 2,
            pltpu.VMEM_SHARED((n_pad, d), jnp.float32),
            [pltpu.SemaphoreType.DMA] * 2,
            [pltpu.SemaphoreType.DMA] * 2,
            [pltpu.SemaphoreType.DMA] * 2,
        ],
    )
    def agg(x_hbm, src_hbm, dst_hbm, et_hbm, norm_hbm, rel_hbm, zeros_hbm,
            out_hbm, src_v, et_v, dst_v, norm_v, xrows_v, relrows_v,
            acc_sh, semi, semx, semr):
        c = lax.axis_index("c")
        s = lax.axis_index("s")
        row0 = s * rows_w

        pltpu.sync_copy(zeros_hbm, acc_sh.at[pl.ds(row0, rows_w)])
        plsc.subcore_barrier()

        base = c * half + s * per_w

        def idx_copies(j, b):
            off = base + j * _CHUNK
            return (
                pltpu.make_async_copy(src_hbm.at[pl.ds(off, _CHUNK)],
                                      src_v[b], semi[b]),
                pltpu.make_async_copy(et_hbm.at[pl.ds(off, _CHUNK)],
                                      et_v[b], semi[b]),
                pltpu.make_async_copy(dst_hbm.at[pl.ds(off, _CHUNK)],
                                      dst_v[b], semi[b]),
                pltpu.make_async_copy(norm_hbm.at[pl.ds(off, _CHUNK)],
                                      norm_v[b], semi[b]),
            )

        def issue_idx(j, b):
            for cp in idx_copies(j, b):
                cp.start()

        def wait_idx(j, b):
            for cp in idx_copies(j, b):
                cp.wait()

        def gather_copies(b):
            return (
                pltpu.make_async_copy(x_hbm.at[src_v[b]], xrows_v[b], semx[b]),
                pltpu.make_async_copy(rel_hbm.at[et_v[b]], relrows_v[b],
                                      semr[b]),
            )

        def issue_gather(b):
            for cp in gather_copies(b):
                cp.start()

        def wait_gather(b):
            for cp in gather_copies(b):
                cp.wait()

        def compute_scatter(b):
            def edge_body(i, carry2):
                nb = plsc.load_gather(norm_v[b],
                                      [jnp.full((16,), i, jnp.int32)])
                for q in range(d // 16):
                    sl = pl.ds(q * 16, 16)
                    xrows_v[b][i, sl] = (xrows_v[b][i, sl]
                                         * relrows_v[b][i, sl] * nb)
                return carry2

            lax.fori_loop(0, _CHUNK, edge_body, 0)
            pltpu.sync_copy(xrows_v[b], acc_sh.at[dst_v[b]], add=True)

        def step(j, b):
            @pl.when(j + 1 < n_chunks)
            def _():
                wait_idx(j + 1, 1 - b)
                issue_gather(1 - b)

            wait_gather(b)
            compute_scatter(b)

            @pl.when(j + 2 < n_chunks)
            def _():
                issue_idx(j + 2, b)

        issue_idx(0, 0)
        wait_idx(0, 0)
        issue_gather(0)
        issue_idx(1, 1)

        def pair_body(jj, carry):
            step(jj * 2, 0)
            step(jj * 2 + 1, 1)
            return carry

        lax.fori_loop(0, n_chunks // 2, pair_body, 0)
        if n_chunks % 2:
            step(n_chunks - 1, 0)

        plsc.subcore_barrier()
        pltpu.sync_copy(acc_sh.at[pl.ds(row0, rows_w)],
                        out_hbm.at[c, pl.ds(row0, rows_w)])

    return agg(x, src, dst, etype, norm, rel, zeros)


_BLK = 2000


def _tc_stage1(acc_ref, x_ref, inw_ref, outw_ref, loopw_ref, wrel_ref,
               looprel_ref, bias_ref, rel_ref, h_ref, stats_ref, relout_ref):
    i = pl.program_id(0)
    hp = lax.Precision.HIGHEST
    f32 = jnp.float32
    h = jnp.dot(acc_ref[0], inw_ref[:], precision=hp,
                preferred_element_type=f32)
    h = h + jnp.dot(acc_ref[1], outw_ref[:], precision=hp,
                    preferred_element_type=f32)
    h = h + jnp.dot(x_ref[:] * looprel_ref[:], loopw_ref[:], precision=hp,
                    preferred_element_type=f32)
    h = h * (1.0 / 3.0) + bias_ref[:]
    h_ref[:] = h

    @pl.when(i == 0)
    def _():
        stats_ref[:] = jnp.zeros_like(stats_ref)
        relout_ref[:] = jnp.dot(rel_ref[:], wrel_ref[:], precision=hp,
                                preferred_element_type=f32)

    stats_ref[0:1, :] += jnp.sum(h, axis=0, keepdims=True)
    stats_ref[1:2, :] += jnp.sum(h * h, axis=0, keepdims=True)


def _tc_stage2(h_ref, stats_ref, gamma_ref, beta_ref, out_ref, *, n):
    mean = stats_ref[0:1, :] * (1.0 / n)
    ex2 = stats_ref[1:2, :] * (1.0 / n)
    var = ex2 - mean * mean
    out_ref[:] = ((h_ref[:] - mean)
                  * (gamma_ref[:] * lax.rsqrt(var + 1e-5)) + beta_ref[:])


def kernel(x, edge_index, rel_repr, edge_type, edge_norm, in_w, out_w, loop_w,
           w_rel, loop_rel, bias, bn_gamma, bn_beta):
    n, d = x.shape
    nr = rel_repr.shape[0]
    src = edge_index[0]
    dst = edge_index[1]
    zeros = jnp.zeros((-(-n // (_NS * 8)) * 8, d), jnp.float32)
    acc = _sc_aggregate(x, src, dst, edge_type, edge_norm, rel_repr, zeros)

    grid = n // _BLK
    const = lambda *_: tuple(0 for _ in range(2))
    h, stats, relout = pl.pallas_call(
        _tc_stage1,
        grid=(grid,),
        in_specs=[
            pl.BlockSpec((2, _BLK, d), lambda i: (0, i, 0)),
            pl.BlockSpec((_BLK, d), lambda i: (i, 0)),
            pl.BlockSpec((d, d), const),
            pl.BlockSpec((d, d), const),
            pl.BlockSpec((d, d), const),
            pl.BlockSpec((d, d), const),
            pl.BlockSpec((1, d), const),
            pl.BlockSpec((1, d), const),
            pl.BlockSpec((nr, d), const),
        ],
        out_specs=[
            pl.BlockSpec((_BLK, d), lambda i: (i, 0)),
            pl.BlockSpec((8, d), const),
            pl.BlockSpec((nr, d), const),
        ],
        out_shape=(jax.ShapeDtypeStruct((n, d), jnp.float32),
                   jax.ShapeDtypeStruct((8, d), jnp.float32),
                   jax.ShapeDtypeStruct((nr, d), jnp.float32)),
    )(acc, x, in_w, out_w, loop_w, w_rel, loop_rel,
      bias.reshape(1, d), rel_repr)

    out = pl.pallas_call(
        functools.partial(_tc_stage2, n=n),
        grid=(grid,),
        in_specs=[
            pl.BlockSpec((_BLK, d), lambda i: (i, 0)),
            pl.BlockSpec((8, d), const),
            pl.BlockSpec((1, d), const),
            pl.BlockSpec((1, d), const),
        ],
        out_specs=pl.BlockSpec((_BLK, d), lambda i: (i, 0)),
        out_shape=jax.ShapeDtypeStruct((n, d), jnp.float32),
    )(h, stats, bn_gamma.reshape(1, d), bn_beta.reshape(1, d))
    return out, relout


# speedup vs baseline: 5.9410x; 1.3475x over previous
import functools

import jax
import jax.numpy as jnp
from jax import lax
from jax.experimental import pallas as pl
from jax.experimental.pallas import tpu as pltpu
from jax.experimental.pallas import tpu_sc as plsc

_NC = 2
_NS = 16
_CHUNK = 80


def _sc_aggregate(x, src, dst, etype, norm, rel, zeros):
    n, d = x.shape
    e = src.shape[0]
    half = e // 2
    per_w = half // _NS
    n_chunks = per_w // _CHUNK
    rows_w = -(-n // (_NS * 8)) * 8
    n_pad = rows_w * _NS

    mesh = plsc.VectorSubcoreMesh(core_axis_name="c", subcore_axis_name="s",
                                  num_cores=_NC, num_subcores=_NS)

    @functools.partial(
        pl.kernel,
        out_type=jax.ShapeDtypeStruct((_NC, n_pad, d), jnp.float32),
        mesh=mesh,
        compiler_params=pltpu.CompilerParams(needs_layout_passes=False),
        scratch_types=[
            [pltpu.VMEM((_CHUNK,), jnp.int32)] * 2,
            [pltpu.VMEM((_CHUNK,), jnp.int32)] * 2,
            [pltpu.VMEM((_CHUNK,), jnp.int32)] * 2,
            [pltpu.VMEM((_CHUNK,), jnp.float32)] * 2,
            [pltpu.VMEM((_CHUNK, d), jnp.float32)] * 2,
            [pltpu.VMEM((_CHUNK, d), jnp.float32)] * 2,
            pltpu.VMEM_SHARED((n_pad, d), jnp.float32),
            [pltpu.SemaphoreType.DMA] * 2,
            [pltpu.SemaphoreType.DMA] * 2,
            [pltpu.SemaphoreType.DMA] * 2,
        ],
    )
    def agg(x_hbm, src_hbm, dst_hbm, et_hbm, norm_hbm, rel_hbm, zeros_hbm,
            out_hbm, src_v, et_v, dst_v, norm_v, xrows_v, relrows_v,
            acc_sh, semi, semx, semr):
        c = lax.axis_index("c")
        s = lax.axis_index("s")
        row0 = s * rows_w

        pltpu.sync_copy(zeros_hbm, acc_sh.at[pl.ds(row0, rows_w)])
        plsc.subcore_barrier()

        base = c * half + s * per_w

        def idx_copies(j, b):
            off = base + j * _CHUNK
            return (
                pltpu.make_async_copy(src_hbm.at[pl.ds(off, _CHUNK)],
                                      src_v[b], semi[b]),
                pltpu.make_async_copy(et_hbm.at[pl.ds(off, _CHUNK)],
                                      et_v[b], semi[b]),
                pltpu.make_async_copy(dst_hbm.at[pl.ds(off, _CHUNK)],
                                      dst_v[b], semi[b]),
                pltpu.make_async_copy(norm_hbm.at[pl.ds(off, _CHUNK)],
                                      norm_v[b], semi[b]),
            )

        def issue_idx(j, b):
            for cp in idx_copies(j, b):
                cp.start()

        def wait_idx(j, b):
            for cp in idx_copies(j, b):
                cp.wait()

        def gather_copies(b):
            return (
                pltpu.make_async_copy(x_hbm.at[src_v[b]], xrows_v[b], semx[b]),
                pltpu.make_async_copy(rel_hbm.at[et_v[b]], relrows_v[b],
                                      semr[b]),
            )

        def issue_gather(b):
            for cp in gather_copies(b):
                cp.start()

        def wait_gather(b):
            for cp in gather_copies(b):
                cp.wait()

        def compute_scatter(b):
            def group_body(g, carry2):
                r0 = g * 16
                norm16 = norm_v[b][pl.ds(r0, 16)]
                for k in range(16):
                    nb = jnp.take_along_axis(
                        norm16, jnp.full((16,), k, jnp.int32), axis=0,
                        mode="promise_in_bounds")
                    for q in range(d // 16):
                        sl = pl.ds(q * 16, 16)
                        xrows_v[b][r0 + k, sl] = (xrows_v[b][r0 + k, sl]
                                                  * relrows_v[b][r0 + k, sl]
                                                  * nb)
                return carry2

            lax.fori_loop(0, _CHUNK // 16, group_body, 0)
            pltpu.sync_copy(xrows_v[b], acc_sh.at[dst_v[b]], add=True)

        def step(j, b):
            @pl.when(j + 1 < n_chunks)
            def _():
                wait_idx(j + 1, 1 - b)
                issue_gather(1 - b)

            wait_gather(b)
            compute_scatter(b)

            @pl.when(j + 2 < n_chunks)
            def _():
                issue_idx(j + 2, b)

        issue_idx(0, 0)
        wait_idx(0, 0)
        issue_gather(0)
        issue_idx(1, 1)

        def pair_body(jj, carry):
            step(jj * 2, 0)
            step(jj * 2 + 1, 1)
            return carry

        lax.fori_loop(0, n_chunks // 2, pair_body, 0)
        if n_chunks % 2:
            step(n_chunks - 1, 0)

        plsc.subcore_barrier()
        pltpu.sync_copy(acc_sh.at[pl.ds(row0, rows_w)],
                        out_hbm.at[c, pl.ds(row0, rows_w)])

    return agg(x, src, dst, etype, norm, rel, zeros)


_BLK = 2000


def _tc_stage1(acc_ref, x_ref, inw_ref, outw_ref, loopw_ref, wrel_ref,
               looprel_ref, bias_ref, rel_ref, h_ref, stats_ref, relout_ref):
    i = pl.program_id(0)
    hp = lax.Precision.HIGHEST
    f32 = jnp.float32
    h = jnp.dot(acc_ref[0], inw_ref[:], precision=hp,
                preferred_element_type=f32)
    h = h + jnp.dot(acc_ref[1], outw_ref[:], precision=hp,
                    preferred_element_type=f32)
    h = h + jnp.dot(x_ref[:] * looprel_ref[:], loopw_ref[:], precision=hp,
                    preferred_element_type=f32)
    h = h * (1.0 / 3.0) + bias_ref[:]
    h_ref[:] = h

    @pl.when(i == 0)
    def _():
        stats_ref[:] = jnp.zeros_like(stats_ref)
        relout_ref[:] = jnp.dot(rel_ref[:], wrel_ref[:], precision=hp,
                                preferred_element_type=f32)

    stats_ref[0:1, :] += jnp.sum(h, axis=0, keepdims=True)
    stats_ref[1:2, :] += jnp.sum(h * h, axis=0, keepdims=True)


def _tc_stage2(h_ref, stats_ref, gamma_ref, beta_ref, out_ref, *, n):
    mean = stats_ref[0:1, :] * (1.0 / n)
    ex2 = stats_ref[1:2, :] * (1.0 / n)
    var = ex2 - mean * mean
    out_ref[:] = ((h_ref[:] - mean)
                  * (gamma_ref[:] * lax.rsqrt(var + 1e-5)) + beta_ref[:])


def kernel(x, edge_index, rel_repr, edge_type, edge_norm, in_w, out_w, loop_w,
           w_rel, loop_rel, bias, bn_gamma, bn_beta):
    n, d = x.shape
    nr = rel_repr.shape[0]
    src = edge_index[0]
    dst = edge_index[1]
    zeros = jnp.zeros((-(-n // (_NS * 8)) * 8, d), jnp.float32)
    acc = _sc_aggregate(x, src, dst, edge_type, edge_norm, rel_repr, zeros)

    grid = n // _BLK
    const = lambda *_: tuple(0 for _ in range(2))
    h, stats, relout = pl.pallas_call(
        _tc_stage1,
        grid=(grid,),
        in_specs=[
            pl.BlockSpec((2, _BLK, d), lambda i: (0, i, 0)),
            pl.BlockSpec((_BLK, d), lambda i: (i, 0)),
            pl.BlockSpec((d, d), const),
            pl.BlockSpec((d, d), const),
            pl.BlockSpec((d, d), const),
            pl.BlockSpec((d, d), const),
            pl.BlockSpec((1, d), const),
            pl.BlockSpec((1, d), const),
            pl.BlockSpec((nr, d), const),
        ],
        out_specs=[
            pl.BlockSpec((_BLK, d), lambda i: (i, 0)),
            pl.BlockSpec((8, d), const),
            pl.BlockSpec((nr, d), const),
        ],
        out_shape=(jax.ShapeDtypeStruct((n, d), jnp.float32),
                   jax.ShapeDtypeStruct((8, d), jnp.float32),
                   jax.ShapeDtypeStruct((nr, d), jnp.float32)),
    )(acc, x, in_w, out_w, loop_w, w_rel, loop_rel,
      bias.reshape(1, d), rel_repr)

    out = pl.pallas_call(
        functools.partial(_tc_stage2, n=n),
        grid=(grid,),
        in_specs=[
            pl.BlockSpec((_BLK, d), lambda i: (i, 0)),
            pl.BlockSpec((8, d), const),
            pl.BlockSpec((1, d), const),
            pl.BlockSpec((1, d), const),
        ],
        out_specs=pl.BlockSpec((_BLK, d), lambda i: (i, 0)),
        out_shape=jax.ShapeDtypeStruct((n, d), jnp.float32),
    )(h, stats, bn_gamma.reshape(1, d), bn_beta.reshape(1, d))
    return out, relout


# speedup vs baseline: 8.0207x; 1.3501x over previous
import functools

import jax
import jax.numpy as jnp
from jax import lax
from jax.experimental import pallas as pl
from jax.experimental.pallas import tpu as pltpu
from jax.experimental.pallas import tpu_sc as plsc

_NC = 2
_NS = 16
_CHUNK = 80


def _sc_aggregate(x, src, dst, etype, norm, rel, zeros):
    n, d = x.shape
    e = src.shape[0]
    half = e // 2
    per_w = half // _NS
    n_chunks = per_w // _CHUNK
    rows_w = -(-n // (_NS * 8)) * 8
    n_pad = rows_w * _NS

    mesh = plsc.VectorSubcoreMesh(core_axis_name="c", subcore_axis_name="s",
                                  num_cores=_NC, num_subcores=_NS)

    @functools.partial(
        pl.kernel,
        out_type=jax.ShapeDtypeStruct((_NC, n_pad, d), jnp.float32),
        mesh=mesh,
        compiler_params=pltpu.CompilerParams(needs_layout_passes=False),
        scratch_types=[
            [pltpu.VMEM((_CHUNK,), jnp.int32)] * 2,
            [pltpu.VMEM((_CHUNK,), jnp.int32)] * 2,
            [pltpu.VMEM((_CHUNK,), jnp.int32)] * 2,
            [pltpu.VMEM((_CHUNK,), jnp.float32)] * 2,
            [pltpu.VMEM((_CHUNK, d), jnp.float32)] * 2,
            [pltpu.VMEM((_CHUNK, d), jnp.float32)] * 2,
            pltpu.VMEM_SHARED((n_pad, d), jnp.float32),
            [pltpu.SemaphoreType.DMA] * 2,
            [pltpu.SemaphoreType.DMA] * 2,
            [pltpu.SemaphoreType.DMA] * 2,
        ],
    )
    def agg(x_hbm, src_hbm, dst_hbm, et_hbm, norm_hbm, rel_hbm, zeros_hbm,
            out_hbm, src_v, et_v, dst_v, norm_v, xrows_v, relrows_v,
            acc_sh, semi, semx, semr):
        c = lax.axis_index("c")
        s = lax.axis_index("s")
        row0 = s * rows_w

        pltpu.sync_copy(zeros_hbm, acc_sh.at[pl.ds(row0, rows_w)])
        plsc.subcore_barrier()

        base = c * half + s * per_w

        def idx_copies(j, b):
            off = base + j * _CHUNK
            return (
                pltpu.make_async_copy(src_hbm.at[pl.ds(off, _CHUNK)],
                                      src_v[b], semi[b]),
                pltpu.make_async_copy(et_hbm.at[pl.ds(off, _CHUNK)],
                                      et_v[b], semi[b]),
                pltpu.make_async_copy(dst_hbm.at[pl.ds(off, _CHUNK)],
                                      dst_v[b], semi[b]),
                pltpu.make_async_copy(norm_hbm.at[pl.ds(off, _CHUNK)],
                                      norm_v[b], semi[b]),
            )

        def issue_idx(j, b):
            for cp in idx_copies(j, b):
                cp.start()

        def wait_idx(j, b):
            for cp in idx_copies(j, b):
                cp.wait()

        def gather_copies(b):
            return (
                pltpu.make_async_copy(x_hbm.at[src_v[b]], xrows_v[b], semx[b]),
                pltpu.make_async_copy(rel_hbm.at[et_v[b]], relrows_v[b],
                                      semr[b]),
            )

        def issue_gather(b):
            for cp in gather_copies(b):
                cp.start()

        def wait_gather(b):
            for cp in gather_copies(b):
                cp.wait()

        def compute_scatter(b):
            @plsc.parallel_loop(0, _CHUNK // 16, 1, unroll=1)
            def group_body(g):
                r0 = g * 16
                norm16 = norm_v[b][pl.ds(r0, 16)]
                for k in range(16):
                    nb = jnp.take_along_axis(
                        norm16, jnp.full((16,), k, jnp.int32), axis=0,
                        mode="promise_in_bounds")
                    for q in range(d // 16):
                        sl = pl.ds(q * 16, 16)
                        xrows_v[b][r0 + k, sl] = (xrows_v[b][r0 + k, sl]
                                                  * relrows_v[b][r0 + k, sl]
                                                  * nb)
            pltpu.sync_copy(xrows_v[b], acc_sh.at[dst_v[b]], add=True)

        def step(j, b):
            @pl.when(j + 1 < n_chunks)
            def _():
                wait_idx(j + 1, 1 - b)
                issue_gather(1 - b)

            wait_gather(b)
            compute_scatter(b)

            @pl.when(j + 2 < n_chunks)
            def _():
                issue_idx(j + 2, b)

        issue_idx(0, 0)
        wait_idx(0, 0)
        issue_gather(0)
        issue_idx(1, 1)

        def pair_body(jj, carry):
            step(jj * 2, 0)
            step(jj * 2 + 1, 1)
            return carry

        lax.fori_loop(0, n_chunks // 2, pair_body, 0)
        if n_chunks % 2:
            step(n_chunks - 1, 0)

        plsc.subcore_barrier()
        pltpu.sync_copy(acc_sh.at[pl.ds(row0, rows_w)],
                        out_hbm.at[c, pl.ds(row0, rows_w)])

    return agg(x, src, dst, etype, norm, rel, zeros)


_BLK = 2000


def _tc_stage1(acc_ref, x_ref, inw_ref, outw_ref, loopw_ref, wrel_ref,
               looprel_ref, bias_ref, rel_ref, h_ref, stats_ref, relout_ref):
    i = pl.program_id(0)
    hp = lax.Precision.HIGHEST
    f32 = jnp.float32
    h = jnp.dot(acc_ref[0], inw_ref[:], precision=hp,
                preferred_element_type=f32)
    h = h + jnp.dot(acc_ref[1], outw_ref[:], precision=hp,
                    preferred_element_type=f32)
    h = h + jnp.dot(x_ref[:] * looprel_ref[:], loopw_ref[:], precision=hp,
                    preferred_element_type=f32)
    h = h * (1.0 / 3.0) + bias_ref[:]
    h_ref[:] = h

    @pl.when(i == 0)
    def _():
        stats_ref[:] = jnp.zeros_like(stats_ref)
        relout_ref[:] = jnp.dot(rel_ref[:], wrel_ref[:], precision=hp,
                                preferred_element_type=f32)

    stats_ref[0:1, :] += jnp.sum(h, axis=0, keepdims=True)
    stats_ref[1:2, :] += jnp.sum(h * h, axis=0, keepdims=True)


def _tc_stage2(h_ref, stats_ref, gamma_ref, beta_ref, out_ref, *, n):
    mean = stats_ref[0:1, :] * (1.0 / n)
    ex2 = stats_ref[1:2, :] * (1.0 / n)
    var = ex2 - mean * mean
    out_ref[:] = ((h_ref[:] - mean)
                  * (gamma_ref[:] * lax.rsqrt(var + 1e-5)) + beta_ref[:])


def kernel(x, edge_index, rel_repr, edge_type, edge_norm, in_w, out_w, loop_w,
           w_rel, loop_rel, bias, bn_gamma, bn_beta):
    n, d = x.shape
    nr = rel_repr.shape[0]
    src = edge_index[0]
    dst = edge_index[1]
    zeros = jnp.zeros((-(-n // (_NS * 8)) * 8, d), jnp.float32)
    acc = _sc_aggregate(x, src, dst, edge_type, edge_norm, rel_repr, zeros)

    grid = n // _BLK
    const = lambda *_: tuple(0 for _ in range(2))
    h, stats, relout = pl.pallas_call(
        _tc_stage1,
        grid=(grid,),
        in_specs=[
            pl.BlockSpec((2, _BLK, d), lambda i: (0, i, 0)),
            pl.BlockSpec((_BLK, d), lambda i: (i, 0)),
            pl.BlockSpec((d, d), const),
            pl.BlockSpec((d, d), const),
            pl.BlockSpec((d, d), const),
            pl.BlockSpec((d, d), const),
            pl.BlockSpec((1, d), const),
            pl.BlockSpec((1, d), const),
            pl.BlockSpec((nr, d), const),
        ],
        out_specs=[
            pl.BlockSpec((_BLK, d), lambda i: (i, 0)),
            pl.BlockSpec((8, d), const),
            pl.BlockSpec((nr, d), const),
        ],
        out_shape=(jax.ShapeDtypeStruct((n, d), jnp.float32),
                   jax.ShapeDtypeStruct((8, d), jnp.float32),
                   jax.ShapeDtypeStruct((nr, d), jnp.float32)),
    )(acc, x, in_w, out_w, loop_w, w_rel, loop_rel,
      bias.reshape(1, d), rel_repr)

    out = pl.pallas_call(
        functools.partial(_tc_stage2, n=n),
        grid=(grid,),
        in_specs=[
            pl.BlockSpec((_BLK, d), lambda i: (i, 0)),
            pl.BlockSpec((8, d), const),
            pl.BlockSpec((1, d), const),
            pl.BlockSpec((1, d), const),
        ],
        out_specs=pl.BlockSpec((_BLK, d), lambda i: (i, 0)),
        out_shape=jax.ShapeDtypeStruct((n, d), jnp.float32),
    )(h, stats, bn_gamma.reshape(1, d), bn_beta.reshape(1, d))
    return out, relout
